# baseline (speedup 1.0000x reference)
import functools

import jax
import jax.numpy as jnp
from jax import lax
from jax.experimental import pallas as pl
from jax.experimental.pallas import tpu as pltpu
from jax.experimental.pallas import tpu_sc as plsc

NUM_LAYERS = 42
NUM_LTYPES = 7
NUM_WTYPES = 7
HIDDEN = 16
BATCH = 16384

NROWS = NUM_LAYERS + NUM_LTYPES + NUM_WTYPES
NROWS_PAD = 64

NC = 2
NS = 16
L = 16
NW = NC * NS
BPW = BATCH // NW
NGROUP = BPW // L
ROWS_PER_SUB = NROWS_PAD // NS


def _sc_body(embt_hbm, lt_hbm, wt_hbm, fcwt_hbm, ow_hbm, fcb_hbm, outb_hbm,
             xa_hbm, xb_hbm, xc_hbm, out_hbm,
             embt_v, fcwt_v, src_v, fcw_v, ow_v, fcb_v, outb_v, prow_v,
             mine_v, tabs_v, bc_v, xa_v, xb_v, xc_v, outt_v, shared, sem):
    cid = lax.axis_index("c")
    sid = lax.axis_index("s")
    wid = sid * NC + cid
    base = wid * BPW
    ck = [jnp.full((L,), k, dtype=jnp.int32) for k in range(HIDDEN)]
    lanes = lax.iota(jnp.int32, L)

    copies = [
        pltpu.async_copy(embt_hbm, embt_v, sem),
        pltpu.async_copy(lt_hbm, src_v.at[pl.ds(NUM_LAYERS, NUM_LTYPES)],
                         sem),
        pltpu.async_copy(wt_hbm,
                         src_v.at[pl.ds(NUM_LAYERS + NUM_LTYPES,
                                        NUM_WTYPES)], sem),
        pltpu.async_copy(fcwt_hbm, fcwt_v, sem),
        pltpu.async_copy(ow_hbm, ow_v, sem),
        pltpu.async_copy(fcb_hbm, fcb_v, sem),
        pltpu.async_copy(outb_hbm, outb_v, sem),
        pltpu.async_copy(xa_hbm.at[pl.ds(base, BPW)], xa_v, sem),
        pltpu.async_copy(xb_hbm.at[pl.ds(base, BPW)], xb_v, sem),
        pltpu.async_copy(xc_hbm.at[pl.ds(base, BPW)], xc_v, sem),
    ]
    for cp in copies:
        cp.wait()

    for r in range(NUM_LAYERS):
        src_v[r, :] = plsc.load_gather(embt_v,
                                       [lanes, jnp.full((L,), r, jnp.int32)])
    for r in range(3 * HIDDEN):
        fcw_v[r, :] = plsc.load_gather(fcwt_v,
                                       [lanes, jnp.full((L,), r, jnp.int32)])

    zvec = lax.shift_right_logical(xa_v[pl.ds(0, L)], 31)

    bias = outb_v[...]
    for k in range(HIDDEN):
        bias = bias + plsc.load_gather(fcb_v, [ck[k] + zvec]) * ow_v[k, :]

    for j in range(ROWS_PER_SUB):
        r = jnp.minimum(sid * ROWS_PER_SUB + j, NROWS - 1)
        koff = jnp.where(
            r < NUM_LAYERS, 0,
            jnp.where(r < NUM_LAYERS + NUM_LTYPES, HIDDEN, 2 * HIDDEN))
        rsplat = jnp.full((L,), r, dtype=jnp.int32)
        acc = jnp.zeros((L,), jnp.float32)
        for k in range(HIDDEN):
            s = plsc.load_gather(src_v, [rsplat, ck[k]])
            acc = acc + s * fcw_v[koff + k, :]
        prow_v[...] = acc
        acc2 = jnp.where(r < NUM_LAYERS, 1.0, 0.0) * bias
        for k in range(HIDDEN):
            s = plsc.load_gather(prow_v, [ck[k] + zvec])
            acc2 = acc2 + s * ow_v[k, :]
        mine_v[j, :] = acc2

    pltpu.sync_copy(mine_v, shared.at[pl.ds(sid * ROWS_PER_SUB,
                                            ROWS_PER_SUB)])
    plsc.subcore_barrier()
    pltpu.sync_copy(shared.at[pl.ds(0, NROWS)], tabs_v)

    for b in range(NUM_LTYPES):
        vb = tabs_v[NUM_LAYERS + b, :]
        for c in range(NUM_WTYPES):
            bc_v[b * NUM_WTYPES + c, :] = (
                vb + tabs_v[NUM_LAYERS + NUM_LTYPES + c, :])

    for g in range(NGROUP):
        s = pl.ds(g * L, L)
        a16 = xa_v[s]
        bc16 = xb_v[s] * NUM_WTYPES + xc_v[s]
        for f in range(HIDDEN):
            outt_v[f, s] = (plsc.load_gather(tabs_v, [a16, ck[f]])
                            + plsc.load_gather(bc_v, [bc16, ck[f]]))

    pltpu.sync_copy(outt_v, out_hbm.at[:, pl.ds(base, BPW)])


@functools.lru_cache(maxsize=1)
def _make_sc():
    mesh = plsc.VectorSubcoreMesh(
        core_axis_name="c", subcore_axis_name="s",
        num_cores=NC, num_subcores=NS)
    return pl.kernel(
        _sc_body,
        out_type=jax.ShapeDtypeStruct((HIDDEN, BATCH), jnp.float32),
        mesh=mesh,
        scratch_types=[
            pltpu.VMEM((HIDDEN, NUM_LAYERS), jnp.float32),
            pltpu.VMEM((HIDDEN, 3 * HIDDEN), jnp.float32),
            pltpu.VMEM((NROWS, HIDDEN), jnp.float32),
            pltpu.VMEM((3 * HIDDEN, HIDDEN), jnp.float32),
            pltpu.VMEM((HIDDEN, HIDDEN), jnp.float32),
            pltpu.VMEM((HIDDEN,), jnp.float32),
            pltpu.VMEM((HIDDEN,), jnp.float32),
            pltpu.VMEM((HIDDEN,), jnp.float32),
            pltpu.VMEM((ROWS_PER_SUB, HIDDEN), jnp.float32),
            pltpu.VMEM((NROWS, HIDDEN), jnp.float32),
            pltpu.VMEM((NUM_LTYPES * NUM_WTYPES, HIDDEN), jnp.float32),
            pltpu.VMEM((BPW,), jnp.int32),
            pltpu.VMEM((BPW,), jnp.int32),
            pltpu.VMEM((BPW,), jnp.int32),
            pltpu.VMEM((HIDDEN, BPW), jnp.float32),
            pltpu.VMEM_SHARED((NROWS_PAD, HIDDEN), jnp.float32),
            pltpu.SemaphoreType.DMA,
        ],
        compiler_params=pltpu.CompilerParams(needs_layout_passes=False),
    )


def kernel(x, emb_table, ltype_table, wtype_table, fc_w, fc_b, out_w, out_b):
    x = x.astype(jnp.int32)
    outt = _make_sc()(emb_table.T, ltype_table, wtype_table, fc_w.T, out_w,
                      fc_b, out_b, x[:, 0], x[:, 1], x[:, 2])
    return outt.T


# speedup vs baseline: 1.1740x; 1.1740x over previous
import functools

import jax
import jax.numpy as jnp
from jax import lax
from jax.experimental import pallas as pl
from jax.experimental.pallas import tpu as pltpu
from jax.experimental.pallas import tpu_sc as plsc

NUM_LAYERS = 42
NUM_LTYPES = 7
NUM_WTYPES = 7
HIDDEN = 16
BATCH = 16384

TROWS = NUM_LAYERS * NUM_LTYPES * NUM_WTYPES
TFLAT = TROWS * HIDDEN

NC = 2
NS = 16
L = 16
NW = NC * NS
BPW = BATCH // NW
NGROUP = BPW // L


def _table_body(emb_ref, lt_ref, wt_ref, fcw_ref, fcb_ref, outw_ref, outb_ref,
                t_ref):
    outw = outw_ref[...]
    w1 = jnp.dot(fcw_ref[0:16, :], outw, preferred_element_type=jnp.float32)
    w2 = jnp.dot(fcw_ref[16:32, :], outw, preferred_element_type=jnp.float32)
    w3 = jnp.dot(fcw_ref[32:48, :], outw, preferred_element_type=jnp.float32)
    bias = (jnp.dot(fcb_ref[...], outw, preferred_element_type=jnp.float32)
            + outb_ref[...])
    a = jnp.dot(emb_ref[...], w1, preferred_element_type=jnp.float32) + bias
    b = jnp.dot(lt_ref[...], w2, preferred_element_type=jnp.float32)
    c = jnp.dot(wt_ref[...], w3, preferred_element_type=jnp.float32)
    t_ref[...] = (a[:, None, None, :] + b[None, :, None, :]
                  + c[None, None, :, :])


def _build_table(emb, lt, wt, fc_w, fc_b, out_w, out_b):
    t4 = pl.pallas_call(
        _table_body,
        out_shape=jax.ShapeDtypeStruct(
            (NUM_LAYERS, NUM_LTYPES, NUM_WTYPES, HIDDEN), jnp.float32),
    )(emb, lt, wt, fc_w, fc_b.reshape(1, HIDDEN), out_w,
      out_b.reshape(1, HIDDEN))
    return t4.reshape(TFLAT)


def _sc_gather_body(t_hbm, xa_hbm, xb_hbm, xc_hbm, out_hbm,
                    tab_v, xa_v, xb_v, xc_v, outt_v):
    wid = lax.axis_index("s") * NC + lax.axis_index("c")
    base = wid * BPW
    pltpu.sync_copy(t_hbm, tab_v)
    pltpu.sync_copy(xa_hbm.at[pl.ds(base, BPW)], xa_v)
    pltpu.sync_copy(xb_hbm.at[pl.ds(base, BPW)], xb_v)
    pltpu.sync_copy(xc_hbm.at[pl.ds(base, BPW)], xc_v)
    for g in range(NGROUP):
        s = pl.ds(g * L, L)
        flat = (xa_v[s] * (NUM_LTYPES * NUM_WTYPES * HIDDEN)
                + xb_v[s] * (NUM_WTYPES * HIDDEN) + xc_v[s] * HIDDEN)
        for f in range(HIDDEN):
            outt_v[f, s] = plsc.load_gather(tab_v, [flat + f])
    pltpu.sync_copy(outt_v, out_hbm.at[:, pl.ds(base, BPW)])


@functools.lru_cache(maxsize=1)
def _make_sc_gather():
    mesh = plsc.VectorSubcoreMesh(
        core_axis_name="c", subcore_axis_name="s",
        num_cores=NC, num_subcores=NS)
    return pl.kernel(
        _sc_gather_body,
        out_type=jax.ShapeDtypeStruct((HIDDEN, BATCH), jnp.float32),
        mesh=mesh,
        scratch_types=[
            pltpu.VMEM((TFLAT,), jnp.float32),
            pltpu.VMEM((BPW,), jnp.int32),
            pltpu.VMEM((BPW,), jnp.int32),
            pltpu.VMEM((BPW,), jnp.int32),
            pltpu.VMEM((HIDDEN, BPW), jnp.float32),
        ],
        compiler_params=pltpu.CompilerParams(needs_layout_passes=False),
    )


def kernel(x, emb_table, ltype_table, wtype_table, fc_w, fc_b, out_w, out_b):
    x = x.astype(jnp.int32)
    t = _build_table(emb_table, ltype_table, wtype_table, fc_w, fc_b,
                     out_w, out_b)
    outt = _make_sc_gather()(t, x[:, 0], x[:, 1], x[:, 2])
    return outt.T
